# baseline (speedup 1.0000x reference)
import functools

import jax
import jax.numpy as jnp
from jax import lax
from jax.experimental import pallas as pl
from jax.experimental.pallas import tpu as pltpu
from jax.experimental.pallas import tpu_sc as plsc

N_NODES = 10000
N_EDGES = 160000
NC, NS = 2, 16
EPS = 1e-5
NF = float(N_NODES)


CH = 128
E_PAD = NS * 80 * CH
N_ACC = 10240
RPS = N_ACC // NS
DW = 128


def _seg_sum(table, src4, dst4, dw=DW):
    K = src4.shape[2]
    mesh = plsc.VectorSubcoreMesh(core_axis_name="c", subcore_axis_name="s",
                                  num_cores=NC, num_subcores=NS)

    NCH = K

    @functools.partial(
        pl.kernel, mesh=mesh,
        out_type=jax.ShapeDtypeStruct((NC, N_ACC, dw), jnp.float32),
        compiler_params=pltpu.CompilerParams(use_tc_tiling_on_sc=False),
        scratch_types=[
            pltpu.VMEM((K, CH), jnp.int32),
            pltpu.VMEM((K, CH), jnp.int32),
            pltpu.VMEM((CH, dw), jnp.float32),
            pltpu.VMEM_SHARED((N_ACC, dw), jnp.float32),
            pltpu.SemaphoreType.DMA,
        ],
    )
    def k(x_hbm, src_hbm, dst_hbm, z_hbm, out_hbm, src_v, dst_v,
          rows_v, acc, sem):
        c = lax.axis_index("c")
        s = lax.axis_index("s")
        pltpu.sync_copy(z_hbm, acc.at[pl.ds(s * RPS, RPS)])
        pltpu.sync_copy(src_hbm.at[c, s], src_v)
        pltpu.sync_copy(dst_hbm.at[c, s], dst_v)
        plsc.subcore_barrier()

        def body(j, carry):
            pltpu.async_copy(x_hbm.at[src_v.at[j]], rows_v, sem).wait()
            pltpu.sync_copy(rows_v, acc.at[dst_v.at[j]], add=True)
            return carry

        lax.fori_loop(0, NCH, body, 0)
        plsc.subcore_barrier()
        pltpu.sync_copy(acc.at[pl.ds(s * RPS, RPS)],
                        out_hbm.at[c, pl.ds(s * RPS, RPS)])

    zeros = jnp.zeros((RPS, dw), jnp.float32)
    return k(table, src4, dst4, zeros)


def _leaky(x, s=0.01):
    return jnp.where(x >= 0, x, s * x)


def _sage_layer(agg2, combine, x, cnt, cs_prev, ss_prev, wlt, wrt, bl,
                emit_cnt=False):
    aw = agg2.shape[2]
    din, dout = wlt.shape
    R = 1000
    NB = N_NODES // R

    def body(a_ref, b_ref, x_ref, cnt_ref, cs_ref, ss_ref, wlt_ref, wrt_ref,
             bl_ref, h_ref, cso_ref, sso_ref, *maybe_cnt):
        i = pl.program_id(0)
        mu = cs_ref[...] / NF
        msq = ss_ref[0, 0] / NF - jnp.sum(mu * mu)
        sc = 1.0 / jnp.sqrt(EPS + msq)
        if combine == "sum":
            agg_full = a_ref[0] + b_ref[0]
            aggb = agg_full[:, :din]
        else:
            aggb = jnp.concatenate([a_ref[0], b_ref[0]], axis=1)
        if emit_cnt:
            cntb = agg_full[:, 18:19]
            maybe_cnt[0][...] = cntb
        else:
            cntb = cnt_ref[...]
        denom = jnp.maximum(cntb, 1.0)
        mean_n = (aggb - cntb * mu) / denom * sc
        xn = (x_ref[...] - mu) * sc
        y = (jnp.dot(mean_n, wlt_ref[...], preferred_element_type=jnp.float32)
             + jnp.dot(xn, wrt_ref[...], preferred_element_type=jnp.float32)
             + bl_ref[...])
        h = _leaky(y)
        h_ref[...] = h

        @pl.when(i == 0)
        def _():
            cso_ref[...] = jnp.zeros_like(cso_ref)
            sso_ref[...] = jnp.zeros_like(sso_ref)

        cso_ref[...] += jnp.sum(h, axis=0, keepdims=True)
        sso_ref[...] += jnp.sum(h * h)[None, None]

    out_specs = [
        pl.BlockSpec((R, dout), lambda i: (i, 0)),
        pl.BlockSpec((1, dout), lambda i: (0, 0)),
        pl.BlockSpec((1, 1), lambda i: (0, 0)),
    ]
    out_shape = [
        jax.ShapeDtypeStruct((N_NODES, dout), jnp.float32),
        jax.ShapeDtypeStruct((1, dout), jnp.float32),
        jax.ShapeDtypeStruct((1, 1), jnp.float32),
    ]
    if emit_cnt:
        out_specs.append(pl.BlockSpec((R, 1), lambda i: (i, 0)))
        out_shape.append(jax.ShapeDtypeStruct((N_NODES, 1), jnp.float32))
    return pl.pallas_call(
        body,
        grid=(NB,),
        in_specs=[
            pl.BlockSpec((1, R, aw), lambda i: (0, i, 0)),
            pl.BlockSpec((1, R, aw), lambda i: (1, i, 0)),
            pl.BlockSpec((R, din), lambda i: (i, 0)),
            pl.BlockSpec((R, 1), lambda i: (i, 0)),
            pl.BlockSpec((1, din), lambda i: (0, 0)),
            pl.BlockSpec((1, 1), lambda i: (0, 0)),
            pl.BlockSpec((din, dout), lambda i: (0, 0)),
            pl.BlockSpec((din, dout), lambda i: (0, 0)),
            pl.BlockSpec((1, dout), lambda i: (0, 0)),
        ],
        out_specs=out_specs,
        out_shape=out_shape,
    )(agg2, agg2, x, cnt, cs_prev, ss_prev, wlt, wrt, bl)


def _t1(h5, cs5, ss5, wgt):
    R = 1000
    NB = N_NODES // R
    GPB = R // 25

    def body(h_ref, cs_ref, ss_ref, w_ref, hg_ref, gm_ref):
        mu = cs_ref[...] / NF
        msq = ss_ref[0, 0] / NF - jnp.sum(mu * mu)
        sc = 1.0 / jnp.sqrt(EPS + msq)
        xn = (h_ref[...] - mu) * sc
        hg = jnp.dot(xn, w_ref[...], preferred_element_type=jnp.float32)
        hg_ref[...] = hg
        rows = lax.broadcasted_iota(jnp.int32, (GPB, R), 0)
        cols = lax.broadcasted_iota(jnp.int32, (GPB, R), 1)
        pmat = jnp.where(cols // 25 == rows, 1.0 / 25.0, 0.0)
        gm_ref[...] = jnp.dot(pmat, hg, preferred_element_type=jnp.float32)

    return pl.pallas_call(
        body,
        grid=(NB,),
        in_specs=[
            pl.BlockSpec((R, 256), lambda i: (i, 0)),
            pl.BlockSpec((1, 256), lambda i: (0, 0)),
            pl.BlockSpec((1, 1), lambda i: (0, 0)),
            pl.BlockSpec((256, 256), lambda i: (0, 0)),
        ],
        out_specs=[
            pl.BlockSpec((R, 256), lambda i: (i, 0)),
            pl.BlockSpec((GPB, 256), lambda i: (i, 0)),
        ],
        out_shape=[
            jax.ShapeDtypeStruct((N_NODES, 256), jnp.float32),
            jax.ShapeDtypeStruct((400, 256), jnp.float32),
        ],
    )(h5, cs5, ss5, wgt)


def _gat_dense(h, as_col, ad_col, gsize, valid):
    M = h.shape[0]
    asrc = jnp.dot(h, as_col, preferred_element_type=jnp.float32)
    adst = jnp.dot(h, ad_col, preferred_element_type=jnp.float32)
    adst_row = jnp.reshape(adst, (1, M))
    lg = _leaky(asrc + adst_row, 0.2)
    ridx = lax.broadcasted_iota(jnp.int32, (M, M), 0)
    cidx = lax.broadcasted_iota(jnp.int32, (M, M), 1)
    mask = (ridx // gsize == cidx // gsize) & (ridx < valid) & (cidx < valid)
    m_col = jnp.max(jnp.where(mask, lg, -1e30), axis=0, keepdims=True)
    e = jnp.where(mask, jnp.exp(lg - m_col), 0.0)
    ssum = jnp.sum(e, axis=0, keepdims=True)
    ediag = jnp.sum(jnp.where(ridx == cidx, e, 0.0), axis=0, keepdims=True)
    denom = 2.0 * ssum - ediag
    agg = lax.dot_general(e, h, (((0,), (0,)), ((), ())),
                          preferred_element_type=jnp.float32)
    num = 2.0 * agg - jnp.reshape(ediag, (M, 1)) * h
    return num / (jnp.reshape(denom, (M, 1)) + 1e-16)


def _mlp3(h, ws, bs):
    for wt, b in zip(ws, bs):
        h = _leaky(jnp.dot(h, wt, preferred_element_type=jnp.float32) + b)
    return h


def _t2(hg0, gm, prm):
    def body(hg_ref, gm_ref,
             g1as_ref, g1ad_ref, g1b_ref,
             l11w0_ref, l11b0_ref, l11w1_ref, l11b1_ref, l11w2_ref, l11b2_ref,
             g2w_ref, g2as_ref, g2ad_ref, g2b_ref,
             l12w0_ref, l12b0_ref, l12w1_ref, l12b1_ref, l12w2_ref, l12b2_ref,
             l2w0_ref, l2b0_ref, l2w1_ref, l2b1_ref, l2w2_ref, l2b2_ref,
             l1w_ref, l1b_ref, l3w_ref, l3b_ref,
             out_ref):
        hg = hg_ref[...]
        out0 = _gat_dense(hg, g1as_ref[...], g1ad_ref[...], 25, 1250)
        out0 = out0 + g1b_ref[...]
        r50 = lax.broadcasted_iota(jnp.int32, (50, 1280), 0)
        c50 = lax.broadcasted_iota(jnp.int32, (50, 1280), 1)
        pmat = jnp.where((c50 // 25 == r50) & (c50 < 1250), 1.0 / 25.0, 0.0)
        d0 = jnp.dot(pmat, out0, preferred_element_type=jnp.float32)
        base = gm_ref[...] + g1b_ref[...]
        r400 = lax.broadcasted_iota(jnp.int32, (400, 50), 0)
        c400 = lax.broadcasted_iota(jnp.int32, (400, 50), 1)
        sel = jnp.where(r400 == c400, 1.0, 0.0)
        base50 = base[0:50, :]
        x7 = base + jnp.dot(sel, d0 - base50, preferred_element_type=jnp.float32)
        x7 = _mlp3(x7, [l11w0_ref[...], l11w1_ref[...], l11w2_ref[...]],
                   [l11b0_ref[...], l11b1_ref[...], l11b2_ref[...]])
        h2 = jnp.dot(x7, g2w_ref[...], preferred_element_type=jnp.float32)
        out2 = _gat_dense(h2, g2as_ref[...], g2ad_ref[...], 50, 400)
        out2 = out2 + g2b_ref[...]
        r8 = lax.broadcasted_iota(jnp.int32, (8, 400), 0)
        c8 = lax.broadcasted_iota(jnp.int32, (8, 400), 1)
        qmat = jnp.where(c8 // 50 == r8, 1.0 / 50.0, 0.0)
        x9 = jnp.dot(qmat, out2, preferred_element_type=jnp.float32)
        x9 = _mlp3(x9, [l12w0_ref[...], l12w1_ref[...], l12w2_ref[...]],
                   [l12b0_ref[...], l12b1_ref[...], l12b2_ref[...]])
        xl = jnp.reshape(x9, (32, 128))
        xl = _mlp3(xl, [l2w0_ref[...], l2w1_ref[...], l2w2_ref[...]],
                   [l2b0_ref[...], l2b1_ref[...], l2b2_ref[...]])
        xl = _leaky(xl)
        x2 = jnp.reshape(xl, (8, 4, 512))
        for o in range(3):
            y_o = (l1w_ref[o, 0] * x2[:, 0, :] + l1w_ref[o, 1] * x2[:, 1, :]
                   + l1w_ref[o, 2] * x2[:, 2, :] + l1w_ref[o, 3] * x2[:, 3, :]
                   + l1b_ref[o])
            y_o = _leaky(y_o)
            f_o = jnp.tanh(jnp.dot(y_o, l3w_ref[...],
                                   preferred_element_type=jnp.float32)
                           + l3b_ref[...]) * 2.0
            out_ref[o] = f_o

    def full(shape):
        return pl.BlockSpec(shape, lambda i: tuple(0 for _ in shape))

    def smem():
        return pl.BlockSpec(memory_space=pltpu.SMEM)

    in_specs = [
        pl.BlockSpec((1280, 256), lambda i: (0, 0)),
        full((400, 256)),
        full((256, 1)), full((256, 1)), full((1, 256)),
        full((256, 256)), full((1, 256)), full((256, 256)), full((1, 256)),
        full((256, 256)), full((1, 256)),
        full((256, 512)), full((512, 1)), full((512, 1)), full((1, 512)),
        full((512, 512)), full((1, 512)), full((512, 512)), full((1, 512)),
        full((512, 512)), full((1, 512)),
        full((128, 64)), full((1, 64)), full((64, 128)), full((1, 128)),
        full((128, 512)), full((1, 512)),
        smem(), smem(),
        full((512, 2048)), full((1, 2048)),
    ]
    p = prm
    return pl.pallas_call(
        body,
        grid=(1,),
        in_specs=in_specs,
        out_specs=pl.BlockSpec((3, 8, 2048), lambda i: (0, 0, 0)),
        out_shape=jax.ShapeDtypeStruct((3, 8, 2048), jnp.float32),
    )(hg0, gm,
      p["gat1_as"].reshape(256, 1), p["gat1_ad"].reshape(256, 1),
      p["gat1_b"].reshape(1, 256),
      p["lin11_W0"].T, p["lin11_b0"].reshape(1, 256),
      p["lin11_W1"].T, p["lin11_b1"].reshape(1, 256),
      p["lin11_W2"].T, p["lin11_b2"].reshape(1, 256),
      p["gat2_W"].T, p["gat2_as"].reshape(512, 1), p["gat2_ad"].reshape(512, 1),
      p["gat2_b"].reshape(1, 512),
      p["lin12_W0"].T, p["lin12_b0"].reshape(1, 512),
      p["lin12_W1"].T, p["lin12_b1"].reshape(1, 512),
      p["lin12_W2"].T, p["lin12_b2"].reshape(1, 512),
      p["lin2_W0"].T, p["lin2_b0"].reshape(1, 64),
      p["lin2_W1"].T, p["lin2_b1"].reshape(1, 128),
      p["lin2_W2"].T, p["lin2_b2"].reshape(1, 512),
      p["lin1_W"], p["lin1_b"],
      p["lin3_W"].T, p["lin3_b"].reshape(1, 2048))


def kernel(x, edge_index, batch_size, num_cross, num_pieces, params):
    p = params
    zero = (jnp.asarray(batch_size) - 8
            + jnp.asarray(num_cross) - 50
            + jnp.asarray(num_pieces) - 25).astype(x.dtype)
    x = x + zero

    pad_e = E_PAD - N_EDGES
    src = jnp.concatenate([edge_index[0], jnp.zeros((pad_e,), jnp.int32)])
    dst = jnp.concatenate([edge_index[1],
                           jnp.full((pad_e,), N_NODES, jnp.int32)])
    kf = E_PAD // (NC * NS * CH)
    srcF = src.reshape(NC, NS, kf, CH)
    dstF = dst.reshape(NC, NS, kf, CH)
    ks = E_PAD // (NS * CH)
    srcS = jnp.stack([2 * src, 2 * src + 1]).reshape(NC, NS, ks, CH)
    dstS = jnp.stack([dst, dst]).reshape(NC, NS, ks, CH)

    ones = jnp.ones((N_NODES, 1), jnp.float32)
    pad = jnp.zeros((N_NODES, 32 - 19), jnp.float32)
    x32 = jnp.concatenate([x, ones, pad], axis=1)
    agg = _seg_sum(x32, srcF, dstF, 32)
    cs = jnp.zeros((1, 18), jnp.float32)
    ss = jnp.full((1, 1), (1.0 - EPS) * NF, jnp.float32)
    cnt0 = jnp.zeros((N_NODES, 1), jnp.float32)
    h, cs, ss, cnt = _sage_layer(agg, "sum",
                                 x, cnt0, cs, ss,
                                 p["conv1_Wl"].T, p["conv1_Wr"].T,
                                 p["conv1_bl"].reshape(1, -1), emit_cnt=True)

    agg = _seg_sum(h, srcF, dstF, 64)
    h, cs, ss = _sage_layer(agg, "sum",
                            h, cnt, cs, ss,
                            p["conv2_Wl"].T, p["conv2_Wr"].T,
                            p["conv2_bl"].reshape(1, -1))

    for name in ["conv21", "conv22", "conv3"]:
        agg = _seg_sum(h.reshape(2 * N_NODES, DW), srcS, dstS)
        h, cs, ss = _sage_layer(agg, "cat",
                                h, cnt, cs, ss,
                                p[name + "_Wl"].T, p[name + "_Wr"].T,
                                p[name + "_bl"].reshape(1, -1))

    hg, gm = _t1(h, cs, ss, p["gat1_W"].T)
    out3 = _t2(hg, gm, p)
    return jnp.transpose(out3, (1, 2, 0))


# speedup vs baseline: 1.0800x; 1.0800x over previous
import functools

import jax
import jax.numpy as jnp
from jax import lax
from jax.experimental import pallas as pl
from jax.experimental.pallas import tpu as pltpu
from jax.experimental.pallas import tpu_sc as plsc

N_NODES = 10000
N_EDGES = 160000
NC, NS = 2, 16
EPS = 1e-5
NF = float(N_NODES)


CH = 128
E_PAD = NS * 80 * CH
N_ACC = 10240
RPS = N_ACC // NS
DW = 128


def _seg_sum(table, src4, dst4, dw=DW):
    K = src4.shape[2]
    mesh = plsc.VectorSubcoreMesh(core_axis_name="c", subcore_axis_name="s",
                                  num_cores=NC, num_subcores=NS)

    NCH = K

    @functools.partial(
        pl.kernel, mesh=mesh,
        out_type=jax.ShapeDtypeStruct((NC, N_ACC, dw), jnp.float32),
        compiler_params=pltpu.CompilerParams(use_tc_tiling_on_sc=False),
        scratch_types=[
            pltpu.VMEM((K, CH), jnp.int32),
            pltpu.VMEM((K, CH), jnp.int32),
            pltpu.VMEM((CH, dw), jnp.float32),
            pltpu.VMEM_SHARED((N_ACC, dw), jnp.float32),
            pltpu.SemaphoreType.DMA,
        ],
    )
    def k(x_hbm, src_hbm, dst_hbm, z_hbm, out_hbm, src_v, dst_v,
          rows_v, acc, sem):
        c = lax.axis_index("c")
        s = lax.axis_index("s")
        pltpu.sync_copy(z_hbm, acc.at[pl.ds(s * RPS, RPS)])
        pltpu.sync_copy(src_hbm.at[c, s], src_v)
        pltpu.sync_copy(dst_hbm.at[c, s], dst_v)
        plsc.subcore_barrier()

        def body(j, carry):
            pltpu.async_copy(x_hbm.at[src_v.at[j]], rows_v, sem).wait()
            pltpu.sync_copy(rows_v, acc.at[dst_v.at[j]], add=True)
            return carry

        lax.fori_loop(0, NCH, body, 0)
        plsc.subcore_barrier()
        pltpu.sync_copy(acc.at[pl.ds(s * RPS, RPS)],
                        out_hbm.at[c, pl.ds(s * RPS, RPS)])

    zeros = jnp.zeros((RPS, dw), jnp.float32)
    return k(table, src4, dst4, zeros)


def _split_cat(h, dh):
    return jnp.concatenate([h[:, :dh], h[:, dh:]], axis=0)


def _leaky(x, s=0.01):
    return jnp.where(x >= 0, x, s * x)


def _sage_layer(agg2, combine, x, cnt, cs_prev, ss_prev, wlt, wrt, bl,
                emit_cnt=False):
    aw = agg2.shape[2]
    din, dout = wlt.shape
    R = 1000
    NB = N_NODES // R

    def body(a_ref, b_ref, x_ref, cnt_ref, cs_ref, ss_ref, wlt_ref, wrt_ref,
             bl_ref, h_ref, cso_ref, sso_ref, *maybe_cnt):
        i = pl.program_id(0)
        mu = cs_ref[...] / NF
        msq = ss_ref[0, 0] / NF - jnp.sum(mu * mu)
        sc = 1.0 / jnp.sqrt(EPS + msq)
        if combine == "sum":
            agg_full = a_ref[0] + b_ref[0]
            aggb = agg_full[:, :din]
        else:
            aggb = jnp.concatenate([a_ref[0], b_ref[0]], axis=1)
        if emit_cnt:
            cntb = agg_full[:, 18:19]
            maybe_cnt[0][...] = cntb
        else:
            cntb = cnt_ref[...]
        denom = jnp.maximum(cntb, 1.0)
        mean_n = (aggb - cntb * mu) / denom * sc
        xn = (x_ref[...] - mu) * sc
        y = (jnp.dot(mean_n, wlt_ref[...], preferred_element_type=jnp.float32)
             + jnp.dot(xn, wrt_ref[...], preferred_element_type=jnp.float32)
             + bl_ref[...])
        h = _leaky(y)
        h_ref[...] = h

        @pl.when(i == 0)
        def _():
            cso_ref[...] = jnp.zeros_like(cso_ref)
            sso_ref[...] = jnp.zeros_like(sso_ref)

        cso_ref[...] += jnp.sum(h, axis=0, keepdims=True)
        sso_ref[...] += jnp.sum(h * h)[None, None]

    out_specs = [
        pl.BlockSpec((R, dout), lambda i: (i, 0)),
        pl.BlockSpec((1, dout), lambda i: (0, 0)),
        pl.BlockSpec((1, 1), lambda i: (0, 0)),
    ]
    out_shape = [
        jax.ShapeDtypeStruct((N_NODES, dout), jnp.float32),
        jax.ShapeDtypeStruct((1, dout), jnp.float32),
        jax.ShapeDtypeStruct((1, 1), jnp.float32),
    ]
    if emit_cnt:
        out_specs.append(pl.BlockSpec((R, 1), lambda i: (i, 0)))
        out_shape.append(jax.ShapeDtypeStruct((N_NODES, 1), jnp.float32))
    return pl.pallas_call(
        body,
        grid=(NB,),
        in_specs=[
            pl.BlockSpec((1, R, aw), lambda i: (0, i, 0)),
            pl.BlockSpec((1, R, aw), lambda i: (1, i, 0)),
            pl.BlockSpec((R, din), lambda i: (i, 0)),
            pl.BlockSpec((R, 1), lambda i: (i, 0)),
            pl.BlockSpec((1, din), lambda i: (0, 0)),
            pl.BlockSpec((1, 1), lambda i: (0, 0)),
            pl.BlockSpec((din, dout), lambda i: (0, 0)),
            pl.BlockSpec((din, dout), lambda i: (0, 0)),
            pl.BlockSpec((1, dout), lambda i: (0, 0)),
        ],
        out_specs=out_specs,
        out_shape=out_shape,
    )(agg2, agg2, x, cnt, cs_prev, ss_prev, wlt, wrt, bl)


def _t1(h5, cs5, ss5, wgt):
    R = 1000
    NB = N_NODES // R
    GPB = R // 25

    def body(h_ref, cs_ref, ss_ref, w_ref, hg_ref, gm_ref):
        mu = cs_ref[...] / NF
        msq = ss_ref[0, 0] / NF - jnp.sum(mu * mu)
        sc = 1.0 / jnp.sqrt(EPS + msq)
        xn = (h_ref[...] - mu) * sc
        hg = jnp.dot(xn, w_ref[...], preferred_element_type=jnp.float32)
        hg_ref[...] = hg
        rows = lax.broadcasted_iota(jnp.int32, (GPB, R), 0)
        cols = lax.broadcasted_iota(jnp.int32, (GPB, R), 1)
        pmat = jnp.where(cols // 25 == rows, 1.0 / 25.0, 0.0)
        gm_ref[...] = jnp.dot(pmat, hg, preferred_element_type=jnp.float32)

    return pl.pallas_call(
        body,
        grid=(NB,),
        in_specs=[
            pl.BlockSpec((R, 256), lambda i: (i, 0)),
            pl.BlockSpec((1, 256), lambda i: (0, 0)),
            pl.BlockSpec((1, 1), lambda i: (0, 0)),
            pl.BlockSpec((256, 256), lambda i: (0, 0)),
        ],
        out_specs=[
            pl.BlockSpec((R, 256), lambda i: (i, 0)),
            pl.BlockSpec((GPB, 256), lambda i: (i, 0)),
        ],
        out_shape=[
            jax.ShapeDtypeStruct((N_NODES, 256), jnp.float32),
            jax.ShapeDtypeStruct((400, 256), jnp.float32),
        ],
    )(h5, cs5, ss5, wgt)


def _gat_dense(h, as_col, ad_col, gsize, valid):
    M = h.shape[0]
    asrc = jnp.dot(h, as_col, preferred_element_type=jnp.float32)
    adst = jnp.dot(h, ad_col, preferred_element_type=jnp.float32)
    adst_row = jnp.reshape(adst, (1, M))
    lg = _leaky(asrc + adst_row, 0.2)
    ridx = lax.broadcasted_iota(jnp.int32, (M, M), 0)
    cidx = lax.broadcasted_iota(jnp.int32, (M, M), 1)
    mask = (ridx // gsize == cidx // gsize) & (ridx < valid) & (cidx < valid)
    m_col = jnp.max(jnp.where(mask, lg, -1e30), axis=0, keepdims=True)
    e = jnp.where(mask, jnp.exp(lg - m_col), 0.0)
    ssum = jnp.sum(e, axis=0, keepdims=True)
    ediag = jnp.sum(jnp.where(ridx == cidx, e, 0.0), axis=0, keepdims=True)
    denom = 2.0 * ssum - ediag
    agg = lax.dot_general(e, h, (((0,), (0,)), ((), ())),
                          preferred_element_type=jnp.float32)
    num = 2.0 * agg - jnp.reshape(ediag, (M, 1)) * h
    return num / (jnp.reshape(denom, (M, 1)) + 1e-16)


def _mlp3(h, ws, bs):
    for wt, b in zip(ws, bs):
        h = _leaky(jnp.dot(h, wt, preferred_element_type=jnp.float32) + b)
    return h


def _t2(hg0, gm, prm):
    def body(hg_ref, gm_ref,
             g1as_ref, g1ad_ref, g1b_ref,
             l11w0_ref, l11b0_ref, l11w1_ref, l11b1_ref, l11w2_ref, l11b2_ref,
             g2w_ref, g2as_ref, g2ad_ref, g2b_ref,
             l12w0_ref, l12b0_ref, l12w1_ref, l12b1_ref, l12w2_ref, l12b2_ref,
             l2w0_ref, l2b0_ref, l2w1_ref, l2b1_ref, l2w2_ref, l2b2_ref,
             l1w_ref, l1b_ref, l3w_ref, l3b_ref,
             out_ref):
        hg = hg_ref[...]
        out0 = _gat_dense(hg, g1as_ref[...], g1ad_ref[...], 25, 1250)
        out0 = out0 + g1b_ref[...]
        r50 = lax.broadcasted_iota(jnp.int32, (50, 1280), 0)
        c50 = lax.broadcasted_iota(jnp.int32, (50, 1280), 1)
        pmat = jnp.where((c50 // 25 == r50) & (c50 < 1250), 1.0 / 25.0, 0.0)
        d0 = jnp.dot(pmat, out0, preferred_element_type=jnp.float32)
        base = gm_ref[...] + g1b_ref[...]
        r400 = lax.broadcasted_iota(jnp.int32, (400, 50), 0)
        c400 = lax.broadcasted_iota(jnp.int32, (400, 50), 1)
        sel = jnp.where(r400 == c400, 1.0, 0.0)
        base50 = base[0:50, :]
        x7 = base + jnp.dot(sel, d0 - base50, preferred_element_type=jnp.float32)
        x7 = _mlp3(x7, [l11w0_ref[...], l11w1_ref[...], l11w2_ref[...]],
                   [l11b0_ref[...], l11b1_ref[...], l11b2_ref[...]])
        h2 = jnp.dot(x7, g2w_ref[...], preferred_element_type=jnp.float32)
        out2 = _gat_dense(h2, g2as_ref[...], g2ad_ref[...], 50, 400)
        out2 = out2 + g2b_ref[...]
        r8 = lax.broadcasted_iota(jnp.int32, (8, 400), 0)
        c8 = lax.broadcasted_iota(jnp.int32, (8, 400), 1)
        qmat = jnp.where(c8 // 50 == r8, 1.0 / 50.0, 0.0)
        x9 = jnp.dot(qmat, out2, preferred_element_type=jnp.float32)
        x9 = _mlp3(x9, [l12w0_ref[...], l12w1_ref[...], l12w2_ref[...]],
                   [l12b0_ref[...], l12b1_ref[...], l12b2_ref[...]])
        xl = jnp.reshape(x9, (32, 128))
        xl = _mlp3(xl, [l2w0_ref[...], l2w1_ref[...], l2w2_ref[...]],
                   [l2b0_ref[...], l2b1_ref[...], l2b2_ref[...]])
        xl = _leaky(xl)
        x2 = jnp.reshape(xl, (8, 4, 512))
        for o in range(3):
            y_o = (l1w_ref[o, 0] * x2[:, 0, :] + l1w_ref[o, 1] * x2[:, 1, :]
                   + l1w_ref[o, 2] * x2[:, 2, :] + l1w_ref[o, 3] * x2[:, 3, :]
                   + l1b_ref[o])
            y_o = _leaky(y_o)
            f_o = jnp.tanh(jnp.dot(y_o, l3w_ref[...],
                                   preferred_element_type=jnp.float32)
                           + l3b_ref[...]) * 2.0
            out_ref[o] = f_o

    def full(shape):
        return pl.BlockSpec(shape, lambda i: tuple(0 for _ in shape))

    def smem():
        return pl.BlockSpec(memory_space=pltpu.SMEM)

    in_specs = [
        pl.BlockSpec((1280, 256), lambda i: (0, 0)),
        full((400, 256)),
        full((256, 1)), full((256, 1)), full((1, 256)),
        full((256, 256)), full((1, 256)), full((256, 256)), full((1, 256)),
        full((256, 256)), full((1, 256)),
        full((256, 512)), full((512, 1)), full((512, 1)), full((1, 512)),
        full((512, 512)), full((1, 512)), full((512, 512)), full((1, 512)),
        full((512, 512)), full((1, 512)),
        full((128, 64)), full((1, 64)), full((64, 128)), full((1, 128)),
        full((128, 512)), full((1, 512)),
        smem(), smem(),
        full((512, 2048)), full((1, 2048)),
    ]
    p = prm
    return pl.pallas_call(
        body,
        grid=(1,),
        in_specs=in_specs,
        out_specs=pl.BlockSpec((3, 8, 2048), lambda i: (0, 0, 0)),
        out_shape=jax.ShapeDtypeStruct((3, 8, 2048), jnp.float32),
    )(hg0, gm,
      p["gat1_as"].reshape(256, 1), p["gat1_ad"].reshape(256, 1),
      p["gat1_b"].reshape(1, 256),
      p["lin11_W0"].T, p["lin11_b0"].reshape(1, 256),
      p["lin11_W1"].T, p["lin11_b1"].reshape(1, 256),
      p["lin11_W2"].T, p["lin11_b2"].reshape(1, 256),
      p["gat2_W"].T, p["gat2_as"].reshape(512, 1), p["gat2_ad"].reshape(512, 1),
      p["gat2_b"].reshape(1, 512),
      p["lin12_W0"].T, p["lin12_b0"].reshape(1, 512),
      p["lin12_W1"].T, p["lin12_b1"].reshape(1, 512),
      p["lin12_W2"].T, p["lin12_b2"].reshape(1, 512),
      p["lin2_W0"].T, p["lin2_b0"].reshape(1, 64),
      p["lin2_W1"].T, p["lin2_b1"].reshape(1, 128),
      p["lin2_W2"].T, p["lin2_b2"].reshape(1, 512),
      p["lin1_W"], p["lin1_b"],
      p["lin3_W"].T, p["lin3_b"].reshape(1, 2048))


def kernel(x, edge_index, batch_size, num_cross, num_pieces, params):
    p = params
    zero = (jnp.asarray(batch_size) - 8
            + jnp.asarray(num_cross) - 50
            + jnp.asarray(num_pieces) - 25).astype(x.dtype)
    x = x + zero

    pad_e = E_PAD - N_EDGES
    src = jnp.concatenate([edge_index[0], jnp.zeros((pad_e,), jnp.int32)])
    dst = jnp.concatenate([edge_index[1],
                           jnp.full((pad_e,), N_NODES, jnp.int32)])
    kf = E_PAD // (NC * NS * CH)
    srcF = src.reshape(NC, NS, kf, CH)
    dstF = dst.reshape(NC, NS, kf, CH)
    ks = E_PAD // (NS * CH)
    srcS = jnp.stack([src, src + N_NODES]).reshape(NC, NS, ks, CH)
    dstS = jnp.stack([dst, dst]).reshape(NC, NS, ks, CH)

    ones = jnp.ones((N_NODES, 1), jnp.float32)
    pad = jnp.zeros((N_NODES, 32 - 19), jnp.float32)
    x32 = jnp.concatenate([x, ones, pad], axis=1)
    agg = _seg_sum(x32, srcF, dstF, 32)
    cs = jnp.zeros((1, 18), jnp.float32)
    ss = jnp.full((1, 1), (1.0 - EPS) * NF, jnp.float32)
    cnt0 = jnp.zeros((N_NODES, 1), jnp.float32)
    h, cs, ss, cnt = _sage_layer(agg, "sum",
                                 x, cnt0, cs, ss,
                                 p["conv1_Wl"].T, p["conv1_Wr"].T,
                                 p["conv1_bl"].reshape(1, -1), emit_cnt=True)

    agg = _seg_sum(h, srcF, dstF, 64)
    h, cs, ss = _sage_layer(agg, "sum",
                            h, cnt, cs, ss,
                            p["conv2_Wl"].T, p["conv2_Wr"].T,
                            p["conv2_bl"].reshape(1, -1))

    for name in ["conv21", "conv22", "conv3"]:
        agg = _seg_sum(_split_cat(h, DW), srcS, dstS)
        h, cs, ss = _sage_layer(agg, "cat",
                                h, cnt, cs, ss,
                                p[name + "_Wl"].T, p[name + "_Wr"].T,
                                p[name + "_bl"].reshape(1, -1))

    hg, gm = _t1(h, cs, ss, p["gat1_W"].T)
    out3 = _t2(hg, gm, p)
    return jnp.transpose(out3, (1, 2, 0))
